# baseline (speedup 1.0000x reference)
import jax
import jax.numpy as jnp
from jax.experimental import pallas as pl
from jax.experimental.pallas import tpu as pltpu

_INTERPRET = False

_N = 32
_E = 160
_H = 4096
_G = 4 * _H
_RT = 2048
_K = _G // _RT


def _graph_kernel(ei_ref, w1_ref, b1_ref, w2_ref, b2_ref,
                  a2_ref, w12_ref, cmat_ref):
    ei = ei_ref[:]
    loop = jax.lax.broadcasted_iota(jnp.int32, (1, _N), 1)
    srcv = jnp.concatenate([ei[0:1, :], loop], axis=1)
    dstv = jnp.concatenate([ei[1:2, :], loop], axis=1)
    nio = jax.lax.broadcasted_iota(jnp.int32, (_N, _E), 0)
    S = (jnp.broadcast_to(srcv, (_N, _E)) == nio).astype(jnp.float32)
    D = (jnp.broadcast_to(dstv, (_N, _E)) == nio).astype(jnp.float32)
    deg = jnp.sum(D, axis=1, keepdims=True)
    dinv = jax.lax.rsqrt(deg)
    wsrc = jnp.sum(S * dinv, axis=0, keepdims=True)
    wdst = jnp.sum(D * dinv, axis=0, keepdims=True)
    Dw = D * (wsrc * wdst)
    A = jax.lax.dot_general(Dw, S, (((1,), (1,)), ((), ())),
                            preferred_element_type=jnp.float32)
    a2_ref[:] = jnp.dot(A, A, preferred_element_type=jnp.float32)
    w12 = jnp.dot(w1_ref[:], w2_ref[:], preferred_element_type=jnp.float32)
    w12_ref[:] = w12
    arow = jnp.sum(A, axis=1, keepdims=True)
    c1 = jnp.dot(b1_ref[:], w2_ref[:], preferred_element_type=jnp.float32)
    cmat_ref[:] = arow * c1 + b2_ref[:]


def _graph_call(edge_index, W1, b1, W2, b2):
    return pl.pallas_call(
        _graph_kernel,
        out_shape=(
            jax.ShapeDtypeStruct((_N, _N), jnp.float32),
            jax.ShapeDtypeStruct((8, 64), jnp.float32),
            jax.ShapeDtypeStruct((_N, 64), jnp.float32),
        ),
        interpret=_INTERPRET,
    )(edge_index, W1, b1, W2, b2)


def _ln_kernel(x_ref, w_ref, b_ref, w12_ref, z_ref):
    xb = x_ref[:]
    mu = jnp.mean(xb, axis=1, keepdims=True)
    var = jnp.mean((xb - mu) ** 2, axis=1, keepdims=True)
    ln = (xb - mu) * jax.lax.rsqrt(var + 1e-5) * w_ref[:] + b_ref[:]
    z_ref[:] = jnp.dot(ln, w12_ref[:], preferred_element_type=jnp.float32)


def _ln_call(x2d, ln_w, ln_b, W12):
    rows = x2d.shape[0]
    R = 1600
    return pl.pallas_call(
        _ln_kernel,
        grid=(rows // R,),
        in_specs=[
            pl.BlockSpec((R, 8), lambda i: (i, 0)),
            pl.BlockSpec((1, 8), lambda i: (0, 0)),
            pl.BlockSpec((1, 8), lambda i: (0, 0)),
            pl.BlockSpec((8, 64), lambda i: (0, 0)),
        ],
        out_specs=pl.BlockSpec((R, 64), lambda i: (i, 0)),
        out_shape=jax.ShapeDtypeStruct((rows, 64), jnp.float32),
        interpret=_INTERPRET,
    )(x2d, ln_w, ln_b, W12)


def _mix_kernel(a2_ref, z_ref, y_ref):
    y_ref[:] = jnp.dot(a2_ref[:], z_ref[:],
                       preferred_element_type=jnp.float32)


def _mix_call(A2, Z2):
    cols = Z2.shape[1]
    C = 6400
    return pl.pallas_call(
        _mix_kernel,
        grid=(cols // C,),
        in_specs=[
            pl.BlockSpec((_N, _N), lambda i: (0, 0)),
            pl.BlockSpec((_N, C), lambda i: (0, i)),
        ],
        out_specs=pl.BlockSpec((_N, C), lambda i: (0, i)),
        out_shape=jax.ShapeDtypeStruct((_N, cols), jnp.float32),
        interpret=_INTERPRET,
    )(A2, Z2)


def _proj_kernel(y_ref, wi_ref, p_ref):
    yb = y_ref[:].astype(jnp.bfloat16)
    wb = wi_ref[:].astype(jnp.bfloat16)
    p_ref[:] = jax.lax.dot_general(yb, wb, (((1,), (1,)), ((), ())),
                                   preferred_element_type=jnp.float32)


def _proj_call(Ybig, Wi):
    rows = Ybig.shape[0]
    RT = 1024
    return pl.pallas_call(
        _proj_kernel,
        grid=(_G // RT,),
        in_specs=[
            pl.BlockSpec((rows, 2048), lambda k: (0, 0)),
            pl.BlockSpec((RT, 2048), lambda k: (k, 0)),
        ],
        out_specs=pl.BlockSpec((rows, RT), lambda k: (0, k)),
        out_shape=jax.ShapeDtypeStruct((rows, _G), jnp.float32),
        interpret=_INTERPRET,
    )(Ybig, Wi)


def _lstm_kernel(p_ref, pb_ref, bi_ref, bh_ref, wh_ref, wfc_ref, bfc_ref,
                 out_ref, gates, cs, hs, hb, tc_s):
    t = pl.program_id(0)
    k = pl.program_id(1)
    T = pl.num_programs(0)

    @pl.when(jnp.logical_and(t == 0, k == 0))
    def _init():
        cs[:] = jnp.zeros_like(cs)
        hb[:] = jnp.zeros_like(hb)

    mm = jax.lax.dot_general(hb[:], wh_ref[:], (((1,), (1,)), ((), ())),
                             preferred_element_type=jnp.float32)
    pre = mm + p_ref[:] + pb_ref[:] + bi_ref[:] + bh_ref[:]
    is_g = jnp.logical_and(k >= 4, k < 6)
    al = jnp.where(is_g, 1.0, 0.5).astype(jnp.float32)
    ga = jnp.where(is_g, 0.0, 0.5).astype(jnp.float32)
    gates[:, pl.ds(k * _RT, _RT)] = al * jnp.tanh(al * pre) + ga

    @pl.when(k == 5)
    def _cell():
        g = gates[:]
        c = g[:, _H:2 * _H] * cs[:] + g[:, 0:_H] * g[:, 2 * _H:3 * _H]
        cs[:] = c
        tc_s[:] = jnp.tanh(c)

    @pl.when(k == _K - 1)
    def _update():
        h = gates[:, 3 * _H:4 * _H] * tc_s[:]
        hs[:] = h
        hb[:] = h.astype(jnp.bfloat16)

    @pl.when(jnp.logical_and(t == T - 1, k == _K - 1))
    def _head():
        hw = hs[:] * wfc_ref[:]
        r = jax.lax.broadcasted_iota(jnp.int32, (_H, _N), 0) // 128
        m = jax.lax.broadcasted_iota(jnp.int32, (_H, _N), 1)
        seg = (r == m).astype(jnp.float32)
        out_ref[:] = jnp.dot(hw, seg,
                             preferred_element_type=jnp.float32) + bfc_ref[:]


def _lstm_call(P, pb, bi, bh, Whb, wfc_t, bfc):
    B = 16
    T = 50
    return pl.pallas_call(
        _lstm_kernel,
        grid=(T, _K),
        in_specs=[
            pl.BlockSpec((B, _RT), lambda t, k: (t, k)),
            pl.BlockSpec((1, _RT), lambda t, k: (0, k)),
            pl.BlockSpec((1, _RT), lambda t, k: (0, k)),
            pl.BlockSpec((1, _RT), lambda t, k: (0, k)),
            pl.BlockSpec((_RT, _H), lambda t, k: (k, 0)),
            pl.BlockSpec((1, _H), lambda t, k: (0, 0)),
            pl.BlockSpec((1, 1), lambda t, k: (0, 0)),
        ],
        out_specs=pl.BlockSpec((B, _N), lambda t, k: (0, 0)),
        out_shape=jax.ShapeDtypeStruct((B, _N), jnp.float32),
        scratch_shapes=[
            pltpu.VMEM((B, _G), jnp.float32),
            pltpu.VMEM((B, _H), jnp.float32),
            pltpu.VMEM((B, _H), jnp.float32),
            pltpu.VMEM((B, _H), jnp.bfloat16),
            pltpu.VMEM((B, _H), jnp.float32),
        ],
        compiler_params=pltpu.CompilerParams(
            dimension_semantics=("arbitrary", "arbitrary")),
        interpret=_INTERPRET,
    )(P, pb, bi, bh, Whb, wfc_t, bfc)


def kernel(x, edge_index, ln_w, ln_b, W1, b1, W2, b2, Wi, Wh, bi, bh, Wfc, bfc):
    B, T, N, F = x.shape

    A2, W12, Cmat = _graph_call(edge_index, W1, b1.reshape(1, 64),
                                W2, b2.reshape(1, 64))

    xT = jnp.transpose(x, (1, 0, 2, 3)).reshape(T * B * N, F)
    Z = _ln_call(xT, ln_w.reshape(1, F), ln_b.reshape(1, F), W12)

    Z2 = Z.reshape(T * B, N, 64).transpose(1, 0, 2).reshape(N, T * B * 64)
    Y2 = _mix_call(A2, Z2)
    Yflat = Y2.reshape(N, T * B, 64).transpose(1, 0, 2).reshape(T * B, N * 64)

    cvec = Cmat.reshape(1, N * 64)
    Ybig = jnp.concatenate(
        [Yflat, cvec, jnp.zeros((7, N * 64), jnp.float32)], axis=0)

    Pbig = _proj_call(Ybig, Wi)
    P, pb = Pbig[:T * B], Pbig[T * B:T * B + 1]

    Whb = Wh.astype(jnp.bfloat16)
    out = _lstm_call(P, pb, bi.reshape(1, _G), bh.reshape(1, _G),
                     Whb, jnp.tile(Wfc, (1, N)), bfc.reshape(1, 1))
    return out.reshape(B, N, 1)


# speedup vs baseline: 1.0076x; 1.0076x over previous
import jax
import jax.numpy as jnp
from jax.experimental import pallas as pl
from jax.experimental.pallas import tpu as pltpu

_INTERPRET = False

_N = 32
_E = 160
_H = 4096
_G = 4 * _H
_RT = 2048
_K = _G // _RT


def _graph_kernel(ei_ref, w1_ref, b1_ref, w2_ref, b2_ref,
                  a2_ref, w12_ref, cmat_ref):
    ei = ei_ref[:]
    loop = jax.lax.broadcasted_iota(jnp.int32, (1, _N), 1)
    srcv = jnp.concatenate([ei[0:1, :], loop], axis=1)
    dstv = jnp.concatenate([ei[1:2, :], loop], axis=1)
    nio = jax.lax.broadcasted_iota(jnp.int32, (_N, _E), 0)
    S = (jnp.broadcast_to(srcv, (_N, _E)) == nio).astype(jnp.float32)
    D = (jnp.broadcast_to(dstv, (_N, _E)) == nio).astype(jnp.float32)
    deg = jnp.sum(D, axis=1, keepdims=True)
    dinv = jax.lax.rsqrt(deg)
    wsrc = jnp.sum(S * dinv, axis=0, keepdims=True)
    wdst = jnp.sum(D * dinv, axis=0, keepdims=True)
    Dw = D * (wsrc * wdst)
    A = jax.lax.dot_general(Dw, S, (((1,), (1,)), ((), ())),
                            preferred_element_type=jnp.float32)
    a2_ref[:] = jnp.dot(A, A, preferred_element_type=jnp.float32)
    w12 = jnp.dot(w1_ref[:], w2_ref[:], preferred_element_type=jnp.float32)
    w12_ref[:] = w12
    arow = jnp.sum(A, axis=1, keepdims=True)
    c1 = jnp.dot(b1_ref[:], w2_ref[:], preferred_element_type=jnp.float32)
    cmat_ref[:] = arow * c1 + b2_ref[:]


def _graph_call(edge_index, W1, b1, W2, b2):
    return pl.pallas_call(
        _graph_kernel,
        out_shape=(
            jax.ShapeDtypeStruct((_N, _N), jnp.float32),
            jax.ShapeDtypeStruct((8, 64), jnp.float32),
            jax.ShapeDtypeStruct((_N, 64), jnp.float32),
        ),
        interpret=_INTERPRET,
    )(edge_index, W1, b1, W2, b2)


def _ln_kernel(x_ref, w_ref, b_ref, w12_ref, z_ref):
    xb = x_ref[:]
    mu = jnp.mean(xb, axis=1, keepdims=True)
    var = jnp.mean((xb - mu) ** 2, axis=1, keepdims=True)
    ln = (xb - mu) * jax.lax.rsqrt(var + 1e-5) * w_ref[:] + b_ref[:]
    z_ref[:] = jnp.dot(ln, w12_ref[:], preferred_element_type=jnp.float32)


def _ln_call(x2d, ln_w, ln_b, W12):
    rows = x2d.shape[0]
    R = 1600
    return pl.pallas_call(
        _ln_kernel,
        grid=(rows // R,),
        in_specs=[
            pl.BlockSpec((R, 8), lambda i: (i, 0)),
            pl.BlockSpec((1, 8), lambda i: (0, 0)),
            pl.BlockSpec((1, 8), lambda i: (0, 0)),
            pl.BlockSpec((8, 64), lambda i: (0, 0)),
        ],
        out_specs=pl.BlockSpec((R, 64), lambda i: (i, 0)),
        out_shape=jax.ShapeDtypeStruct((rows, 64), jnp.float32),
        interpret=_INTERPRET,
    )(x2d, ln_w, ln_b, W12)


def _mix_kernel(a2_ref, z_ref, y_ref):
    y_ref[:] = jnp.dot(a2_ref[:], z_ref[:],
                       preferred_element_type=jnp.float32)


def _mix_call(A2, Z2):
    cols = Z2.shape[1]
    C = 6400
    return pl.pallas_call(
        _mix_kernel,
        grid=(cols // C,),
        in_specs=[
            pl.BlockSpec((_N, _N), lambda i: (0, 0)),
            pl.BlockSpec((_N, C), lambda i: (0, i)),
        ],
        out_specs=pl.BlockSpec((_N, C), lambda i: (0, i)),
        out_shape=jax.ShapeDtypeStruct((_N, cols), jnp.float32),
        interpret=_INTERPRET,
    )(A2, Z2)


def _proj_kernel(y_ref, wi_ref, p_ref):
    yb = y_ref[:].astype(jnp.bfloat16)
    wb = wi_ref[:].astype(jnp.bfloat16)
    p_ref[:] = jax.lax.dot_general(yb, wb, (((1,), (1,)), ((), ())),
                                   preferred_element_type=jnp.float32)


def _proj_call(Ybig, Wi):
    rows = Ybig.shape[0]
    RT = 1024
    return pl.pallas_call(
        _proj_kernel,
        grid=(_G // RT,),
        in_specs=[
            pl.BlockSpec((rows, 2048), lambda k: (0, 0)),
            pl.BlockSpec((RT, 2048), lambda k: (k, 0)),
        ],
        out_specs=pl.BlockSpec((rows, RT), lambda k: (0, k)),
        out_shape=jax.ShapeDtypeStruct((rows, _G), jnp.float32),
        interpret=_INTERPRET,
    )(Ybig, Wi)


def _lstm_kernel(p_ref, pb_ref, bi_ref, bh_ref, wh_ref, wfc_ref, bfc_ref,
                 out_ref, gates, cs, hs, hb, tc_s):
    t = pl.program_id(0)
    k = pl.program_id(1)
    T = pl.num_programs(0)

    @pl.when(jnp.logical_and(t == 0, k == 0))
    def _init():
        cs[:] = jnp.zeros_like(cs)
        hb[:] = jnp.zeros_like(hb)

    mm = jax.lax.dot_general(hb[:], wh_ref[0], (((1,), (0,)), ((), ())),
                             preferred_element_type=jnp.float32)
    pre = mm + p_ref[:] + pb_ref[:] + bi_ref[:] + bh_ref[:]
    is_g = jnp.logical_and(k >= 4, k < 6)
    al = jnp.where(is_g, 1.0, 0.5).astype(jnp.float32)
    ga = jnp.where(is_g, 0.0, 0.5).astype(jnp.float32)
    gates[:, pl.ds(k * _RT, _RT)] = al * jnp.tanh(al * pre) + ga

    @pl.when(k == 5)
    def _cell():
        g = gates[:]
        c = g[:, _H:2 * _H] * cs[:] + g[:, 0:_H] * g[:, 2 * _H:3 * _H]
        cs[:] = c
        tc_s[:] = jnp.tanh(c)

    @pl.when(k == _K - 1)
    def _update():
        h = gates[:, 3 * _H:4 * _H] * tc_s[:]
        hs[:] = h
        hb[:] = h.astype(jnp.bfloat16)

    @pl.when(jnp.logical_and(t == T - 1, k == _K - 1))
    def _head():
        hw = hs[:] * wfc_ref[:]
        r = jax.lax.broadcasted_iota(jnp.int32, (_H, _N), 0) // 128
        m = jax.lax.broadcasted_iota(jnp.int32, (_H, _N), 1)
        seg = (r == m).astype(jnp.float32)
        out_ref[:] = jnp.dot(hw, seg,
                             preferred_element_type=jnp.float32) + bfc_ref[:]


def _lstm_call(P, pb, bi, bh, Whb, wfc_t, bfc):
    B = 16
    T = 50
    return pl.pallas_call(
        _lstm_kernel,
        grid=(T, _K),
        in_specs=[
            pl.BlockSpec((B, _RT), lambda t, k: (t, k)),
            pl.BlockSpec((1, _RT), lambda t, k: (0, k)),
            pl.BlockSpec((1, _RT), lambda t, k: (0, k)),
            pl.BlockSpec((1, _RT), lambda t, k: (0, k)),
            pl.BlockSpec((1, _H, _RT), lambda t, k: (k, 0, 0)),
            pl.BlockSpec((1, _H), lambda t, k: (0, 0)),
            pl.BlockSpec((1, 1), lambda t, k: (0, 0)),
        ],
        out_specs=pl.BlockSpec((B, _N), lambda t, k: (0, 0)),
        out_shape=jax.ShapeDtypeStruct((B, _N), jnp.float32),
        scratch_shapes=[
            pltpu.VMEM((B, _G), jnp.float32),
            pltpu.VMEM((B, _H), jnp.float32),
            pltpu.VMEM((B, _H), jnp.float32),
            pltpu.VMEM((B, _H), jnp.bfloat16),
            pltpu.VMEM((B, _H), jnp.float32),
        ],
        compiler_params=pltpu.CompilerParams(
            dimension_semantics=("arbitrary", "arbitrary")),
        interpret=_INTERPRET,
    )(P, pb, bi, bh, Whb, wfc_t, bfc)


def kernel(x, edge_index, ln_w, ln_b, W1, b1, W2, b2, Wi, Wh, bi, bh, Wfc, bfc):
    B, T, N, F = x.shape

    A2, W12, Cmat = _graph_call(edge_index, W1, b1.reshape(1, 64),
                                W2, b2.reshape(1, 64))

    xT = jnp.transpose(x, (1, 0, 2, 3)).reshape(T * B * N, F)
    Z = _ln_call(xT, ln_w.reshape(1, F), ln_b.reshape(1, F), W12)

    Z2 = Z.reshape(T * B, N, 64).transpose(1, 0, 2).reshape(N, T * B * 64)
    Y2 = _mix_call(A2, Z2)
    Yflat = Y2.reshape(N, T * B, 64).transpose(1, 0, 2).reshape(T * B, N * 64)

    cvec = Cmat.reshape(1, N * 64)
    Ybig = jnp.concatenate(
        [Yflat, cvec, jnp.zeros((7, N * 64), jnp.float32)], axis=0)

    Pbig = _proj_call(Ybig, Wi)
    P, pb = Pbig[:T * B], Pbig[T * B:T * B + 1]

    Whb = jnp.transpose(Wh.astype(jnp.bfloat16).reshape(_K, _RT, _H),
                        (0, 2, 1))
    out = _lstm_call(P, pb, bi.reshape(1, _G), bh.reshape(1, _G),
                     Whb, jnp.tile(Wfc, (1, N)), bfc.reshape(1, 1))
    return out.reshape(B, N, 1)


# speedup vs baseline: 1.0084x; 1.0008x over previous
import jax
import jax.numpy as jnp
from jax.experimental import pallas as pl
from jax.experimental.pallas import tpu as pltpu

_INTERPRET = False

_N = 32
_E = 160
_H = 4096
_G = 4 * _H
_RT = 2048
_K = _G // _RT


def _graph_kernel(ei_ref, w1_ref, b1_ref, w2_ref, b2_ref,
                  a2_ref, w12_ref, cmat_ref):
    ei = ei_ref[:]
    loop = jax.lax.broadcasted_iota(jnp.int32, (1, _N), 1)
    srcv = jnp.concatenate([ei[0:1, :], loop], axis=1)
    dstv = jnp.concatenate([ei[1:2, :], loop], axis=1)
    nio = jax.lax.broadcasted_iota(jnp.int32, (_N, _E), 0)
    S = (jnp.broadcast_to(srcv, (_N, _E)) == nio).astype(jnp.float32)
    D = (jnp.broadcast_to(dstv, (_N, _E)) == nio).astype(jnp.float32)
    deg = jnp.sum(D, axis=1, keepdims=True)
    dinv = jax.lax.rsqrt(deg)
    wsrc = jnp.sum(S * dinv, axis=0, keepdims=True)
    wdst = jnp.sum(D * dinv, axis=0, keepdims=True)
    Dw = D * (wsrc * wdst)
    A = jax.lax.dot_general(Dw, S, (((1,), (1,)), ((), ())),
                            preferred_element_type=jnp.float32)
    a2_ref[:] = jnp.dot(A, A, preferred_element_type=jnp.float32)
    w12 = jnp.dot(w1_ref[:], w2_ref[:], preferred_element_type=jnp.float32)
    w12_ref[:] = w12
    arow = jnp.sum(A, axis=1, keepdims=True)
    c1 = jnp.dot(b1_ref[:], w2_ref[:], preferred_element_type=jnp.float32)
    cmat_ref[:] = arow * c1 + b2_ref[:]


def _graph_call(edge_index, W1, b1, W2, b2):
    return pl.pallas_call(
        _graph_kernel,
        out_shape=(
            jax.ShapeDtypeStruct((_N, _N), jnp.float32),
            jax.ShapeDtypeStruct((8, 64), jnp.float32),
            jax.ShapeDtypeStruct((_N, 64), jnp.float32),
        ),
        interpret=_INTERPRET,
    )(edge_index, W1, b1, W2, b2)


def _ln_kernel(x_ref, w_ref, b_ref, w12_ref, z_ref):
    xb = x_ref[:]
    mu = jnp.mean(xb, axis=1, keepdims=True)
    var = jnp.mean((xb - mu) ** 2, axis=1, keepdims=True)
    ln = (xb - mu) * jax.lax.rsqrt(var + 1e-5) * w_ref[:] + b_ref[:]
    z_ref[:] = jnp.dot(ln, w12_ref[:], preferred_element_type=jnp.float32)


def _ln_call(x2d, ln_w, ln_b, W12):
    rows = x2d.shape[0]
    R = 1600
    return pl.pallas_call(
        _ln_kernel,
        grid=(rows // R,),
        in_specs=[
            pl.BlockSpec((R, 8), lambda i: (i, 0)),
            pl.BlockSpec((1, 8), lambda i: (0, 0)),
            pl.BlockSpec((1, 8), lambda i: (0, 0)),
            pl.BlockSpec((8, 64), lambda i: (0, 0)),
        ],
        out_specs=pl.BlockSpec((R, 64), lambda i: (i, 0)),
        out_shape=jax.ShapeDtypeStruct((rows, 64), jnp.float32),
        interpret=_INTERPRET,
    )(x2d, ln_w, ln_b, W12)


def _mix_kernel(a2_ref, z_ref, y_ref):
    y_ref[:] = jnp.dot(a2_ref[:], z_ref[:],
                       preferred_element_type=jnp.float32)


def _mix_call(A2, Z2):
    cols = Z2.shape[1]
    C = 6400
    return pl.pallas_call(
        _mix_kernel,
        grid=(cols // C,),
        in_specs=[
            pl.BlockSpec((_N, _N), lambda i: (0, 0)),
            pl.BlockSpec((_N, C), lambda i: (0, i)),
        ],
        out_specs=pl.BlockSpec((_N, C), lambda i: (0, i)),
        out_shape=jax.ShapeDtypeStruct((_N, cols), jnp.float32),
        interpret=_INTERPRET,
    )(A2, Z2)


def _proj_kernel(y_ref, wi_ref, p_ref):
    yb = y_ref[:].astype(jnp.bfloat16)
    wb = wi_ref[:].astype(jnp.bfloat16)
    p_ref[:] = jax.lax.dot_general(yb, wb, (((1,), (1,)), ((), ())),
                                   preferred_element_type=jnp.float32)


def _proj_call(Ybig, Wi):
    rows = Ybig.shape[0]
    RT = 1024
    return pl.pallas_call(
        _proj_kernel,
        grid=(_G // RT,),
        in_specs=[
            pl.BlockSpec((rows, 2048), lambda k: (0, 0)),
            pl.BlockSpec((RT, 2048), lambda k: (k, 0)),
        ],
        out_specs=pl.BlockSpec((rows, RT), lambda k: (0, k)),
        out_shape=jax.ShapeDtypeStruct((rows, _G), jnp.float32),
        interpret=_INTERPRET,
    )(Ybig, Wi)


def _lstm_kernel(p_ref, gb_ref, wh_ref, wfc_ref, bfc_ref,
                 out_ref, gates, cs, hs, hb, tc_s):
    t = pl.program_id(0)
    k = pl.program_id(1)
    T = pl.num_programs(0)

    @pl.when(jnp.logical_and(t == 0, k == 0))
    def _init():
        cs[:] = jnp.zeros_like(cs)
        hb[:] = jnp.zeros_like(hb)

    mm = jax.lax.dot_general(hb[:], wh_ref[0], (((1,), (0,)), ((), ())),
                             preferred_element_type=jnp.float32)
    sl = pl.ds(k * _RT, _RT)
    pre = mm + p_ref[:, sl] + gb_ref[:, sl]
    is_g = jnp.logical_and(k >= 4, k < 6)
    al = jnp.where(is_g, 1.0, 0.5).astype(jnp.float32)
    ga = jnp.where(is_g, 0.0, 0.5).astype(jnp.float32)
    gates[:, sl] = al * jnp.tanh(al * pre) + ga

    @pl.when(k == 5)
    def _cell():
        g = gates[:]
        c = g[:, _H:2 * _H] * cs[:] + g[:, 0:_H] * g[:, 2 * _H:3 * _H]
        cs[:] = c
        tc_s[:] = jnp.tanh(c)

    @pl.when(k == _K - 1)
    def _update():
        h = gates[:, 3 * _H:4 * _H] * tc_s[:]
        hs[:] = h
        hb[:] = h.astype(jnp.bfloat16)

    @pl.when(jnp.logical_and(t == T - 1, k == _K - 1))
    def _head():
        hw = hs[:] * wfc_ref[:]
        r = jax.lax.broadcasted_iota(jnp.int32, (_H, _N), 0) // 128
        m = jax.lax.broadcasted_iota(jnp.int32, (_H, _N), 1)
        seg = (r == m).astype(jnp.float32)
        out_ref[:] = jnp.dot(hw, seg,
                             preferred_element_type=jnp.float32) + bfc_ref[:]


def _lstm_call(P, gb, Whb, wfc_t, bfc):
    B = 16
    T = 50
    return pl.pallas_call(
        _lstm_kernel,
        grid=(T, _K),
        in_specs=[
            pl.BlockSpec((B, _G), lambda t, k: (t, 0)),
            pl.BlockSpec((1, _G), lambda t, k: (0, 0)),
            pl.BlockSpec((1, _H, _RT), lambda t, k: (k, 0, 0)),
            pl.BlockSpec((1, _H), lambda t, k: (0, 0)),
            pl.BlockSpec((1, 1), lambda t, k: (0, 0)),
        ],
        out_specs=pl.BlockSpec((B, _N), lambda t, k: (0, 0)),
        out_shape=jax.ShapeDtypeStruct((B, _N), jnp.float32),
        scratch_shapes=[
            pltpu.VMEM((B, _G), jnp.float32),
            pltpu.VMEM((B, _H), jnp.float32),
            pltpu.VMEM((B, _H), jnp.float32),
            pltpu.VMEM((B, _H), jnp.bfloat16),
            pltpu.VMEM((B, _H), jnp.float32),
        ],
        compiler_params=pltpu.CompilerParams(
            dimension_semantics=("arbitrary", "arbitrary")),
        interpret=_INTERPRET,
    )(P, gb, Whb, wfc_t, bfc)


def kernel(x, edge_index, ln_w, ln_b, W1, b1, W2, b2, Wi, Wh, bi, bh, Wfc, bfc):
    B, T, N, F = x.shape

    A2, W12, Cmat = _graph_call(edge_index, W1, b1.reshape(1, 64),
                                W2, b2.reshape(1, 64))

    xT = jnp.transpose(x, (1, 0, 2, 3)).reshape(T * B * N, F)
    Z = _ln_call(xT, ln_w.reshape(1, F), ln_b.reshape(1, F), W12)

    Z2 = Z.reshape(T * B, N, 64).transpose(1, 0, 2).reshape(N, T * B * 64)
    Y2 = _mix_call(A2, Z2)
    Yflat = Y2.reshape(N, T * B, 64).transpose(1, 0, 2).reshape(T * B, N * 64)

    cvec = Cmat.reshape(1, N * 64)
    Ybig = jnp.concatenate(
        [Yflat, cvec, jnp.zeros((7, N * 64), jnp.float32)], axis=0)

    Pbig = _proj_call(Ybig, Wi)
    P = Pbig[:T * B]
    gb = Pbig[T * B:T * B + 1] + bi.reshape(1, _G) + bh.reshape(1, _G)

    Whb = jnp.transpose(Wh.astype(jnp.bfloat16).reshape(_K, _RT, _H),
                        (0, 2, 1))
    out = _lstm_call(P, gb, Whb, jnp.tile(Wfc, (1, N)), bfc.reshape(1, 1))
    return out.reshape(B, N, 1)


# speedup vs baseline: 1.0225x; 1.0140x over previous
import jax
import jax.numpy as jnp
from jax.experimental import pallas as pl
from jax.experimental.pallas import tpu as pltpu

_INTERPRET = False

_N = 32
_E = 160
_H = 4096
_G = 4 * _H
_RT = 2048
_K = _G // _RT


def _graph_kernel(ei_ref, w1_ref, b1_ref, w2_ref, b2_ref,
                  a2_ref, w12_ref, cmat_ref):
    ei = ei_ref[:]
    loop = jax.lax.broadcasted_iota(jnp.int32, (1, _N), 1)
    srcv = jnp.concatenate([ei[0:1, :], loop], axis=1)
    dstv = jnp.concatenate([ei[1:2, :], loop], axis=1)
    nio = jax.lax.broadcasted_iota(jnp.int32, (_N, _E), 0)
    S = (jnp.broadcast_to(srcv, (_N, _E)) == nio).astype(jnp.float32)
    D = (jnp.broadcast_to(dstv, (_N, _E)) == nio).astype(jnp.float32)
    deg = jnp.sum(D, axis=1, keepdims=True)
    dinv = jax.lax.rsqrt(deg)
    wsrc = jnp.sum(S * dinv, axis=0, keepdims=True)
    wdst = jnp.sum(D * dinv, axis=0, keepdims=True)
    Dw = D * (wsrc * wdst)
    A = jax.lax.dot_general(Dw, S, (((1,), (1,)), ((), ())),
                            preferred_element_type=jnp.float32)
    a2_ref[:] = jnp.dot(A, A, preferred_element_type=jnp.float32)
    w12 = jnp.dot(w1_ref[:], w2_ref[:], preferred_element_type=jnp.float32)
    w12_ref[:] = w12
    arow = jnp.sum(A, axis=1, keepdims=True)
    c1 = jnp.dot(b1_ref[:], w2_ref[:], preferred_element_type=jnp.float32)
    cmat_ref[:] = arow * c1 + b2_ref[:]


def _graph_call(edge_index, W1, b1, W2, b2):
    return pl.pallas_call(
        _graph_kernel,
        out_shape=(
            jax.ShapeDtypeStruct((_N, _N), jnp.float32),
            jax.ShapeDtypeStruct((8, 64), jnp.float32),
            jax.ShapeDtypeStruct((_N, 64), jnp.float32),
        ),
        interpret=_INTERPRET,
    )(edge_index, W1, b1, W2, b2)


def _ln_kernel(x_ref, w_ref, b_ref, w12_ref, z_ref):
    xb = x_ref[:]
    mu = jnp.mean(xb, axis=1, keepdims=True)
    var = jnp.mean((xb - mu) ** 2, axis=1, keepdims=True)
    ln = (xb - mu) * jax.lax.rsqrt(var + 1e-5) * w_ref[:] + b_ref[:]
    z_ref[:] = jnp.dot(ln, w12_ref[:], preferred_element_type=jnp.float32)


def _ln_call(x2d, ln_w, ln_b, W12):
    rows = x2d.shape[0]
    R = 1600
    return pl.pallas_call(
        _ln_kernel,
        grid=(rows // R,),
        in_specs=[
            pl.BlockSpec((R, 8), lambda i: (i, 0)),
            pl.BlockSpec((1, 8), lambda i: (0, 0)),
            pl.BlockSpec((1, 8), lambda i: (0, 0)),
            pl.BlockSpec((8, 64), lambda i: (0, 0)),
        ],
        out_specs=pl.BlockSpec((R, 64), lambda i: (i, 0)),
        out_shape=jax.ShapeDtypeStruct((rows, 64), jnp.float32),
        interpret=_INTERPRET,
    )(x2d, ln_w, ln_b, W12)


def _mix_kernel(a2_ref, z_ref, y_ref):
    y_ref[:] = jnp.dot(a2_ref[:], z_ref[:],
                       preferred_element_type=jnp.float32)


def _mix_call(A2, Z2):
    cols = Z2.shape[1]
    C = 6400
    return pl.pallas_call(
        _mix_kernel,
        grid=(cols // C,),
        in_specs=[
            pl.BlockSpec((_N, _N), lambda i: (0, 0)),
            pl.BlockSpec((_N, C), lambda i: (0, i)),
        ],
        out_specs=pl.BlockSpec((_N, C), lambda i: (0, i)),
        out_shape=jax.ShapeDtypeStruct((_N, cols), jnp.float32),
        interpret=_INTERPRET,
    )(A2, Z2)


def _proj_kernel(y_ref, wi_ref, p_ref):
    yb = y_ref[:].astype(jnp.bfloat16)
    wb = wi_ref[:].astype(jnp.bfloat16)
    p_ref[:] = jax.lax.dot_general(yb, wb, (((1,), (1,)), ((), ())),
                                   preferred_element_type=jnp.float32)


def _proj_call(Ybig, Wi):
    rows = Ybig.shape[0]
    RT = 1024
    return pl.pallas_call(
        _proj_kernel,
        grid=(_G // RT,),
        in_specs=[
            pl.BlockSpec((rows, 2048), lambda k: (0, 0)),
            pl.BlockSpec((RT, 2048), lambda k: (k, 0)),
        ],
        out_specs=pl.BlockSpec((rows, RT), lambda k: (0, k)),
        out_shape=jax.ShapeDtypeStruct((rows, _G), jnp.float32),
        interpret=_INTERPRET,
    )(Ybig, Wi)


def _lstm_kernel(p_ref, gb_ref, wh_ref, wfc_ref, bfc_ref,
                 out_ref, gates, cs, hs, hb, tc_s):
    t = pl.program_id(0)
    k = pl.program_id(1)
    T = pl.num_programs(0)

    @pl.when(jnp.logical_and(t == 0, k == 0))
    def _init():
        cs[:] = jnp.zeros_like(cs)
        hb[:] = jnp.zeros_like(hb)

    mm = jax.lax.dot_general(hb[:], wh_ref[0], (((1,), (0,)), ((), ())),
                             preferred_element_type=jnp.float32)
    sl = pl.ds(k * _RT, _RT)
    pre = mm + p_ref[:, sl] + gb_ref[:, sl]
    is_g = jnp.logical_and(k >= 4, k < 6)
    al = jnp.where(is_g, 1.0, 0.5).astype(jnp.float32)
    ga = jnp.where(is_g, 0.0, 0.5).astype(jnp.float32)
    gates[:, sl] = al * jnp.tanh(al * pre) + ga

    @pl.when(k == 5)
    def _cell():
        g = gates[:]
        c = g[:, _H:2 * _H] * cs[:] + g[:, 0:_H] * g[:, 2 * _H:3 * _H]
        cs[:] = c
        tc_s[:] = jnp.tanh(c)

    @pl.when(k == _K - 1)
    def _update():
        h = gates[:, 3 * _H:4 * _H] * tc_s[:]
        hs[:] = h
        hb[:] = h.astype(jnp.bfloat16)

    @pl.when(jnp.logical_and(t == T - 1, k == _K - 1))
    def _head():
        hw = hs[:] * wfc_ref[:]
        r = jax.lax.broadcasted_iota(jnp.int32, (_H, _N), 0) // 128
        m = jax.lax.broadcasted_iota(jnp.int32, (_H, _N), 1)
        seg = (r == m).astype(jnp.float32)
        out_ref[:] = jnp.dot(hw, seg,
                             preferred_element_type=jnp.float32) + bfc_ref[:]


def _lstm_call(P, gb, Whb, wfc_t, bfc):
    B = 16
    T = 50
    return pl.pallas_call(
        _lstm_kernel,
        grid=(T, _K),
        in_specs=[
            pl.BlockSpec((B, _G), lambda t, k: (t, 0)),
            pl.BlockSpec((1, _G), lambda t, k: (0, 0)),
            pl.BlockSpec((1, _H, _RT), lambda t, k: (k, 0, 0)),
            pl.BlockSpec((1, _H), lambda t, k: (0, 0)),
            pl.BlockSpec((1, 1), lambda t, k: (0, 0)),
        ],
        out_specs=pl.BlockSpec((B, _N), lambda t, k: (0, 0)),
        out_shape=jax.ShapeDtypeStruct((B, _N), jnp.float32),
        scratch_shapes=[
            pltpu.VMEM((B, _G), jnp.float32),
            pltpu.VMEM((B, _H), jnp.float32),
            pltpu.VMEM((B, _H), jnp.float32),
            pltpu.VMEM((B, _H), jnp.bfloat16),
            pltpu.VMEM((B, _H), jnp.float32),
        ],
        compiler_params=pltpu.CompilerParams(
            dimension_semantics=("arbitrary", "arbitrary")),
        interpret=_INTERPRET,
    )(P, gb, Whb, wfc_t, bfc)


def kernel(x, edge_index, ln_w, ln_b, W1, b1, W2, b2, Wi, Wh, bi, bh, Wfc, bfc):
    B, T, N, F = x.shape

    A2, W12, Cmat = _graph_call(edge_index, W1, b1.reshape(1, 64),
                                W2, b2.reshape(1, 64))

    xT = jnp.transpose(x, (1, 0, 2, 3)).reshape(T * B * N, F)
    Z = _ln_call(xT, ln_w.reshape(1, F), ln_b.reshape(1, F), W12)

    Z2 = Z.reshape(T * B, N, 64).transpose(1, 0, 2).reshape(N, T * B * 64)
    Y2 = _mix_call(A2, Z2)
    Yflat = Y2.reshape(N, T * B, 64).transpose(1, 0, 2).reshape(T * B, N * 64)

    cvec = Cmat.reshape(1, N * 64)
    Ybig = jnp.concatenate(
        [Yflat, cvec, jnp.zeros((7, N * 64), jnp.float32)], axis=0)

    Pbig = _proj_call(Ybig, Wi)
    gb = Pbig[T * B:T * B + 1] + bi.reshape(1, _G) + bh.reshape(1, _G)

    Whb = jnp.transpose(Wh.astype(jnp.bfloat16).reshape(_K, _RT, _H),
                        (0, 2, 1))
    out = _lstm_call(Pbig, gb, Whb, jnp.tile(Wfc, (1, N)), bfc.reshape(1, 1))
    return out.reshape(B, N, 1)


# speedup vs baseline: 1.4709x; 1.4385x over previous
import functools

import jax
import jax.numpy as jnp
import numpy as np
from jax import lax
from jax.experimental import pallas as pl
from jax.experimental.pallas import tpu as pltpu
from jax.experimental.shard_map import shard_map
from jax.sharding import Mesh, NamedSharding, PartitionSpec as P_

_INTERPRET = False

_N = 32
_E = 160
_H = 4096
_G = 4 * _H
_RT = 2048
_K = _G // _RT


def _graph_kernel(ei_ref, w1_ref, b1_ref, w2_ref, b2_ref,
                  a2_ref, w12_ref, cmat_ref):
    ei = ei_ref[:]
    loop = jax.lax.broadcasted_iota(jnp.int32, (1, _N), 1)
    srcv = jnp.concatenate([ei[0:1, :], loop], axis=1)
    dstv = jnp.concatenate([ei[1:2, :], loop], axis=1)
    nio = jax.lax.broadcasted_iota(jnp.int32, (_N, _E), 0)
    S = (jnp.broadcast_to(srcv, (_N, _E)) == nio).astype(jnp.float32)
    D = (jnp.broadcast_to(dstv, (_N, _E)) == nio).astype(jnp.float32)
    deg = jnp.sum(D, axis=1, keepdims=True)
    dinv = jax.lax.rsqrt(deg)
    wsrc = jnp.sum(S * dinv, axis=0, keepdims=True)
    wdst = jnp.sum(D * dinv, axis=0, keepdims=True)
    Dw = D * (wsrc * wdst)
    A = jax.lax.dot_general(Dw, S, (((1,), (1,)), ((), ())),
                            preferred_element_type=jnp.float32)
    a2_ref[:] = jnp.dot(A, A, preferred_element_type=jnp.float32)
    w12 = jnp.dot(w1_ref[:], w2_ref[:], preferred_element_type=jnp.float32)
    w12_ref[:] = w12
    arow = jnp.sum(A, axis=1, keepdims=True)
    c1 = jnp.dot(b1_ref[:], w2_ref[:], preferred_element_type=jnp.float32)
    cmat_ref[:] = arow * c1 + b2_ref[:]


def _graph_call(edge_index, W1, b1, W2, b2):
    return pl.pallas_call(
        _graph_kernel,
        out_shape=(
            jax.ShapeDtypeStruct((_N, _N), jnp.float32),
            jax.ShapeDtypeStruct((8, 64), jnp.float32),
            jax.ShapeDtypeStruct((_N, 64), jnp.float32),
        ),
        interpret=_INTERPRET,
    )(edge_index, W1, b1, W2, b2)


def _ln_kernel(x_ref, w_ref, b_ref, w12_ref, z_ref):
    xb = x_ref[:]
    mu = jnp.mean(xb, axis=1, keepdims=True)
    var = jnp.mean((xb - mu) ** 2, axis=1, keepdims=True)
    ln = (xb - mu) * jax.lax.rsqrt(var + 1e-5) * w_ref[:] + b_ref[:]
    z_ref[:] = jnp.dot(ln, w12_ref[:], preferred_element_type=jnp.float32)


def _ln_call(x2d, ln_w, ln_b, W12):
    rows = x2d.shape[0]
    R = 1600
    return pl.pallas_call(
        _ln_kernel,
        grid=(rows // R,),
        in_specs=[
            pl.BlockSpec((R, 8), lambda i: (i, 0)),
            pl.BlockSpec((1, 8), lambda i: (0, 0)),
            pl.BlockSpec((1, 8), lambda i: (0, 0)),
            pl.BlockSpec((8, 64), lambda i: (0, 0)),
        ],
        out_specs=pl.BlockSpec((R, 64), lambda i: (i, 0)),
        out_shape=jax.ShapeDtypeStruct((rows, 64), jnp.float32),
        interpret=_INTERPRET,
    )(x2d, ln_w, ln_b, W12)


def _mix_kernel(a2_ref, z_ref, y_ref):
    y_ref[:] = jnp.dot(a2_ref[:], z_ref[:],
                       preferred_element_type=jnp.float32)


def _mix_call(A2, Z2):
    cols = Z2.shape[1]
    C = 6400
    return pl.pallas_call(
        _mix_kernel,
        grid=(cols // C,),
        in_specs=[
            pl.BlockSpec((_N, _N), lambda i: (0, 0)),
            pl.BlockSpec((_N, C), lambda i: (0, i)),
        ],
        out_specs=pl.BlockSpec((_N, C), lambda i: (0, i)),
        out_shape=jax.ShapeDtypeStruct((_N, cols), jnp.float32),
        interpret=_INTERPRET,
    )(A2, Z2)


def _proj_kernel(y_ref, wi_ref, p_ref):
    yb = y_ref[:].astype(jnp.bfloat16)
    wb = wi_ref[:].astype(jnp.bfloat16)
    p_ref[:] = jax.lax.dot_general(yb, wb, (((1,), (1,)), ((), ())),
                                   preferred_element_type=jnp.float32)


def _proj_call(Ybig, Wi, permute):
    rows = Ybig.shape[0]
    RT = 1024
    if permute:
        omap = lambda k: (0, (k % 4 // 2) * 8 + (k // 4) * 2 + k % 2)
    else:
        omap = lambda k: (0, k)
    return pl.pallas_call(
        _proj_kernel,
        grid=(_G // RT,),
        in_specs=[
            pl.BlockSpec((rows, 2048), lambda k: (0, 0)),
            pl.BlockSpec((RT, 2048), lambda k: (k, 0)),
        ],
        out_specs=pl.BlockSpec((rows, RT), omap),
        out_shape=jax.ShapeDtypeStruct((rows, _G), jnp.float32),
        interpret=_INTERPRET,
    )(Ybig, Wi)


def _lstm_kernel(p_ref, gb_ref, wh_ref, wfc_ref, bfc_ref,
                 out_ref, gates, cs, hs, hb, tc_s):
    t = pl.program_id(0)
    k = pl.program_id(1)
    T = pl.num_programs(0)

    @pl.when(jnp.logical_and(t == 0, k == 0))
    def _init():
        cs[:] = jnp.zeros_like(cs)
        hb[:] = jnp.zeros_like(hb)

    mm = jax.lax.dot_general(hb[:], wh_ref[0], (((1,), (0,)), ((), ())),
                             preferred_element_type=jnp.float32)
    sl = pl.ds(k * _RT, _RT)
    pre = mm + p_ref[:, sl] + gb_ref[:, sl]
    is_g = jnp.logical_and(k >= 4, k < 6)
    al = jnp.where(is_g, 1.0, 0.5).astype(jnp.float32)
    ga = jnp.where(is_g, 0.0, 0.5).astype(jnp.float32)
    gates[:, sl] = al * jnp.tanh(al * pre) + ga

    @pl.when(k == 5)
    def _cell():
        g = gates[:]
        c = g[:, _H:2 * _H] * cs[:] + g[:, 0:_H] * g[:, 2 * _H:3 * _H]
        cs[:] = c
        tc_s[:] = jnp.tanh(c)

    @pl.when(k == _K - 1)
    def _update():
        h = gates[:, 3 * _H:4 * _H] * tc_s[:]
        hs[:] = h
        hb[:] = h.astype(jnp.bfloat16)

    @pl.when(jnp.logical_and(t == T - 1, k == _K - 1))
    def _head():
        hw = hs[:] * wfc_ref[:]
        r = jax.lax.broadcasted_iota(jnp.int32, (_H, _N), 0) // 128
        m = jax.lax.broadcasted_iota(jnp.int32, (_H, _N), 1)
        seg = (r == m).astype(jnp.float32)
        out_ref[:] = jnp.dot(hw, seg,
                             preferred_element_type=jnp.float32) + bfc_ref[:]


def _lstm_call(P, gb, Whb, wfc_t, bfc):
    B = 16
    T = 50
    return pl.pallas_call(
        _lstm_kernel,
        grid=(T, _K),
        in_specs=[
            pl.BlockSpec((B, _G), lambda t, k: (t, 0)),
            pl.BlockSpec((1, _G), lambda t, k: (0, 0)),
            pl.BlockSpec((1, _H, _RT), lambda t, k: (k, 0, 0)),
            pl.BlockSpec((1, _H), lambda t, k: (0, 0)),
            pl.BlockSpec((1, 1), lambda t, k: (0, 0)),
        ],
        out_specs=pl.BlockSpec((B, _N), lambda t, k: (0, 0)),
        out_shape=jax.ShapeDtypeStruct((B, _N), jnp.float32),
        scratch_shapes=[
            pltpu.VMEM((B, _G), jnp.float32),
            pltpu.VMEM((B, _H), jnp.float32),
            pltpu.VMEM((B, _H), jnp.float32),
            pltpu.VMEM((B, _H), jnp.bfloat16),
            pltpu.VMEM((B, _H), jnp.float32),
        ],
        compiler_params=pltpu.CompilerParams(
            dimension_semantics=("arbitrary", "arbitrary")),
        interpret=_INTERPRET,
    )(P, gb, Whb, wfc_t, bfc)


def _lstm2_kernel(cid_ref, p_ref, gb_ref, wh_ref, wfc_ref, bfc_ref,
                  out_ref, gates, cs, hs, hbf, tc_s, send_sem, recv_sem):
    t = pl.program_id(0)
    k = pl.program_id(1)
    T = pl.num_programs(0)
    cid = cid_ref[0]
    peer = 1 - cid
    HH = _H // 2

    @pl.when(jnp.logical_and(t == 0, k == 0))
    def _init():
        bar = pltpu.get_barrier_semaphore()
        pltpu.semaphore_signal(bar, 1, device_id=peer,
                               device_id_type=pltpu.DeviceIdType.LOGICAL)
        pltpu.semaphore_wait(bar, 1)
        cs[:] = jnp.zeros_like(cs)
        hbf[:] = jnp.zeros_like(hbf)

    wh = wh_ref[0]
    mm = (jax.lax.dot_general(hbf[0], wh[0:HH], (((1,), (0,)), ((), ())),
                              preferred_element_type=jnp.float32)
          + jax.lax.dot_general(hbf[1], wh[HH:2 * HH],
                                (((1,), (0,)), ((), ())),
                                preferred_element_type=jnp.float32))
    sl = pl.ds(k * HH, HH)
    pre = mm + p_ref[:, sl] + gb_ref[:, sl]
    al = jnp.where(k == 2, 1.0, 0.5).astype(jnp.float32)
    ga = jnp.where(k == 2, 0.0, 0.5).astype(jnp.float32)
    gates[:, sl] = al * jnp.tanh(al * pre) + ga

    @pl.when(k == 2)
    def _cell():
        g = gates[:]
        c = g[:, HH:2 * HH] * cs[:] + g[:, 0:HH] * g[:, 2 * HH:3 * HH]
        cs[:] = c
        tc_s[:] = jnp.tanh(c)

    @pl.when(k == 3)
    def _update():
        h = gates[:, 3 * HH:4 * HH] * tc_s[:]
        hs[:] = h
        hbf[cid] = h.astype(jnp.bfloat16)

        @pl.when(t < T - 1)
        def _exchange():
            rc = pltpu.make_async_remote_copy(
                hbf.at[cid], hbf.at[cid], send_sem, recv_sem,
                device_id=peer,
                device_id_type=pltpu.DeviceIdType.LOGICAL)
            rc.start()
            rc.wait_send()
            rc.wait_recv()

    @pl.when(jnp.logical_and(t == T - 1, k == 3))
    def _head():
        hw = hs[:] * wfc_ref[:]
        r = jax.lax.broadcasted_iota(jnp.int32, (HH, 16), 0) // 128
        m = jax.lax.broadcasted_iota(jnp.int32, (HH, 16), 1)
        seg = (r == m).astype(jnp.float32)
        out_ref[:] = jnp.dot(hw, seg,
                             preferred_element_type=jnp.float32) + bfc_ref[:]


def _lstm2_local(cid, Pl, gbl, Whl, wfcl, bfc):
    B = 16
    T = 50
    GH = _G // 2
    return pl.pallas_call(
        _lstm2_kernel,
        grid=(T, 4),
        in_specs=[
            pl.BlockSpec(memory_space=pltpu.SMEM),
            pl.BlockSpec((B, GH), lambda t, k: (t, 0)),
            pl.BlockSpec((1, GH), lambda t, k: (0, 0)),
            pl.BlockSpec((1, _H, _H // 2), lambda t, k: (k, 0, 0)),
            pl.BlockSpec((1, _H // 2), lambda t, k: (0, 0)),
            pl.BlockSpec((1, 1), lambda t, k: (0, 0)),
        ],
        out_specs=pl.BlockSpec((B, 16), lambda t, k: (0, 0)),
        out_shape=jax.ShapeDtypeStruct((B, 16), jnp.float32),
        scratch_shapes=[
            pltpu.VMEM((B, GH), jnp.float32),
            pltpu.VMEM((B, _H // 2), jnp.float32),
            pltpu.VMEM((B, _H // 2), jnp.float32),
            pltpu.VMEM((2, B, _H // 2), jnp.bfloat16),
            pltpu.VMEM((B, _H // 2), jnp.float32),
            pltpu.SemaphoreType.DMA,
            pltpu.SemaphoreType.DMA,
        ],
        compiler_params=pltpu.CompilerParams(
            dimension_semantics=("arbitrary", "arbitrary"),
            collective_id=0),
        interpret=_INTERPRET,
    )(cid, Pl, gbl, Whl, wfcl, bfc)


def _lstm2_call(Pbig, gb, Wh5, wfc_t, bfc, mesh):
    def local_fn(Pl, gbl, Whl, wfcl, bfcl):
        cid = lax.axis_index("x").astype(jnp.int32).reshape(1)
        return _lstm2_local(cid, Pl, gbl, Whl[0], wfcl, bfcl)

    return shard_map(
        local_fn, mesh=mesh,
        in_specs=(P_(None, "x"), P_(None, "x"), P_("x"), P_(None, "x"),
                  P_(None, None)),
        out_specs=P_(None, "x"),
        check_rep=False,
    )(Pbig, gb, Wh5, wfc_t, bfc)


def _rep(mesh, fn, nout):
    outs = tuple(P_() for _ in range(nout))

    def wrap(*args):
        return shard_map(fn, mesh=mesh,
                         in_specs=tuple(P_() for _ in args),
                         out_specs=outs if nout > 1 else P_(),
                         check_rep=False)(*args)

    return wrap


def kernel(x, edge_index, ln_w, ln_b, W1, b1, W2, b2, Wi, Wh, bi, bh, Wfc, bfc):
    B, T, N, F = x.shape

    devs = jax.devices()
    two_core = len(devs) >= 2 and devs[0].platform == "tpu"
    if two_core:
        mesh = Mesh(np.array(devs[:2]), ("x",))
        graph_c = _rep(mesh, _graph_call, 3)
        ln_c = _rep(mesh, _ln_call, 1)
        mix_c = _rep(mesh, _mix_call, 1)
        proj_c = _rep(mesh, lambda y, wi: _proj_call(y, wi, True), 1)
    else:
        mesh = None
        graph_c, ln_c, mix_c = _graph_call, _ln_call, _mix_call
        proj_c = lambda y, wi: _proj_call(y, wi, False)

    A2, W12, Cmat = graph_c(edge_index, W1, b1.reshape(1, 64),
                            W2, b2.reshape(1, 64))

    xT = jnp.transpose(x, (1, 0, 2, 3)).reshape(T * B * N, F)
    Z = ln_c(xT, ln_w.reshape(1, F), ln_b.reshape(1, F), W12)

    Z2 = Z.reshape(T * B, N, 64).transpose(1, 0, 2).reshape(N, T * B * 64)
    Y2 = mix_c(A2, Z2)
    Yflat = Y2.reshape(N, T * B, 64).transpose(1, 0, 2).reshape(T * B, N * 64)

    cvec = Cmat.reshape(1, N * 64)
    Ybig = jnp.concatenate(
        [Yflat, cvec, jnp.zeros((7, N * 64), jnp.float32)], axis=0)

    bsum = bi.reshape(1, _G) + bh.reshape(1, _G)
    wfc_t = jnp.tile(Wfc, (1, N))
    bfc2 = bfc.reshape(1, 1)

    if two_core:
        Pbig = proj_c(Ybig, Wi)
        gbp = bsum.reshape(4, 2, _H // 2).transpose(1, 0, 2).reshape(1, _G)
        gb = Pbig[T * B:T * B + 1] + gbp
        Wh5 = jnp.transpose(
            Wh.astype(jnp.bfloat16).reshape(4, 2, _H // 2, _H), (1, 0, 3, 2))
        out = _lstm2_call(Pbig, gb, Wh5, wfc_t, bfc2, mesh)
    else:
        Pbig = proj_c(Ybig, Wi)
        gb = Pbig[T * B:T * B + 1] + bsum
        Whb = jnp.transpose(Wh.astype(jnp.bfloat16).reshape(_K, _RT, _H),
                            (0, 2, 1))
        out = _lstm_call(Pbig, gb, Whb, wfc_t, bfc2)
    return out.reshape(B, N, 1)


# speedup vs baseline: 1.5757x; 1.0713x over previous
import functools

import jax
import jax.numpy as jnp
import numpy as np
from jax import lax
from jax.experimental import pallas as pl
from jax.experimental.pallas import tpu as pltpu
from jax.experimental.shard_map import shard_map
from jax.sharding import Mesh, NamedSharding, PartitionSpec as P_

_INTERPRET = False

_N = 32
_E = 160
_H = 4096
_G = 4 * _H
_RT = 2048
_K = _G // _RT


def _graph_kernel(ei_ref, w1_ref, b1_ref, w2_ref, b2_ref,
                  a2_ref, w12_ref, cmat_ref):
    ei = ei_ref[:]
    loop = jax.lax.broadcasted_iota(jnp.int32, (1, _N), 1)
    srcv = jnp.concatenate([ei[0:1, :], loop], axis=1)
    dstv = jnp.concatenate([ei[1:2, :], loop], axis=1)
    nio = jax.lax.broadcasted_iota(jnp.int32, (_N, _E), 0)
    S = (jnp.broadcast_to(srcv, (_N, _E)) == nio).astype(jnp.float32)
    D = (jnp.broadcast_to(dstv, (_N, _E)) == nio).astype(jnp.float32)
    deg = jnp.sum(D, axis=1, keepdims=True)
    dinv = jax.lax.rsqrt(deg)
    wsrc = jnp.sum(S * dinv, axis=0, keepdims=True)
    wdst = jnp.sum(D * dinv, axis=0, keepdims=True)
    Dw = D * (wsrc * wdst)
    A = jax.lax.dot_general(Dw, S, (((1,), (1,)), ((), ())),
                            preferred_element_type=jnp.float32)
    a2_ref[:] = jnp.dot(A, A, preferred_element_type=jnp.float32)
    w12 = jnp.dot(w1_ref[:], w2_ref[:], preferred_element_type=jnp.float32)
    w12_ref[:] = w12
    arow = jnp.sum(A, axis=1, keepdims=True)
    c1 = jnp.dot(b1_ref[:], w2_ref[:], preferred_element_type=jnp.float32)
    cmat_ref[:] = arow * c1 + b2_ref[:]


def _graph_call(edge_index, W1, b1, W2, b2):
    return pl.pallas_call(
        _graph_kernel,
        out_shape=(
            jax.ShapeDtypeStruct((_N, _N), jnp.float32),
            jax.ShapeDtypeStruct((8, 64), jnp.float32),
            jax.ShapeDtypeStruct((_N, 64), jnp.float32),
        ),
        interpret=_INTERPRET,
    )(edge_index, W1, b1, W2, b2)


def _ln_kernel(x_ref, w_ref, b_ref, w12_ref, z_ref):
    xb = x_ref[:]
    mu = jnp.mean(xb, axis=1, keepdims=True)
    var = jnp.mean((xb - mu) ** 2, axis=1, keepdims=True)
    ln = (xb - mu) * jax.lax.rsqrt(var + 1e-5) * w_ref[:] + b_ref[:]
    z_ref[:] = jnp.dot(ln, w12_ref[:], preferred_element_type=jnp.float32)


def _ln_call(x2d, ln_w, ln_b, W12):
    rows = x2d.shape[0]
    R = 1600
    return pl.pallas_call(
        _ln_kernel,
        grid=(rows // R,),
        in_specs=[
            pl.BlockSpec((R, 8), lambda i: (i, 0)),
            pl.BlockSpec((1, 8), lambda i: (0, 0)),
            pl.BlockSpec((1, 8), lambda i: (0, 0)),
            pl.BlockSpec((8, 64), lambda i: (0, 0)),
        ],
        out_specs=pl.BlockSpec((R, 64), lambda i: (i, 0)),
        out_shape=jax.ShapeDtypeStruct((rows, 64), jnp.float32),
        interpret=_INTERPRET,
    )(x2d, ln_w, ln_b, W12)


def _mix_kernel(a2_ref, z_ref, y_ref):
    y_ref[:] = jnp.dot(a2_ref[:], z_ref[:],
                       preferred_element_type=jnp.float32)


def _mix_call(A2, Z2):
    cols = Z2.shape[1]
    C = 6400
    return pl.pallas_call(
        _mix_kernel,
        grid=(cols // C,),
        in_specs=[
            pl.BlockSpec((_N, _N), lambda i: (0, 0)),
            pl.BlockSpec((_N, C), lambda i: (0, i)),
        ],
        out_specs=pl.BlockSpec((_N, C), lambda i: (0, i)),
        out_shape=jax.ShapeDtypeStruct((_N, cols), jnp.float32),
        interpret=_INTERPRET,
    )(A2, Z2)


def _proj_kernel(y_ref, wi_ref, p_ref):
    yb = y_ref[:].astype(jnp.bfloat16)
    wb = wi_ref[:].astype(jnp.bfloat16)
    p_ref[:] = jax.lax.dot_general(yb, wb, (((1,), (1,)), ((), ())),
                                   preferred_element_type=jnp.float32)


def _proj_call(Ybig, Wi, permute):
    rows = Ybig.shape[0]
    RT = 1024
    if permute:
        omap = lambda k: (0, (k % 4 // 2) * 8 + (k // 4) * 2 + k % 2)
    else:
        omap = lambda k: (0, k)
    return pl.pallas_call(
        _proj_kernel,
        grid=(_G // RT,),
        in_specs=[
            pl.BlockSpec((rows, 2048), lambda k: (0, 0)),
            pl.BlockSpec((RT, 2048), lambda k: (k, 0)),
        ],
        out_specs=pl.BlockSpec((rows, RT), omap),
        out_shape=jax.ShapeDtypeStruct((rows, _G), jnp.float32),
        interpret=_INTERPRET,
    )(Ybig, Wi)


def _lstm_kernel(p_ref, gb_ref, wh_ref, wfc_ref, bfc_ref,
                 out_ref, gates, cs, hs, hb, tc_s):
    t = pl.program_id(0)
    k = pl.program_id(1)
    T = pl.num_programs(0)

    @pl.when(jnp.logical_and(t == 0, k == 0))
    def _init():
        cs[:] = jnp.zeros_like(cs)
        hb[:] = jnp.zeros_like(hb)

    mm = jax.lax.dot_general(hb[:], wh_ref[0], (((1,), (0,)), ((), ())),
                             preferred_element_type=jnp.float32)
    sl = pl.ds(k * _RT, _RT)
    pre = mm + p_ref[:, sl] + gb_ref[:, sl]
    is_g = jnp.logical_and(k >= 4, k < 6)
    al = jnp.where(is_g, 1.0, 0.5).astype(jnp.float32)
    ga = jnp.where(is_g, 0.0, 0.5).astype(jnp.float32)
    gates[:, sl] = al * jnp.tanh(al * pre) + ga

    @pl.when(k == 5)
    def _cell():
        g = gates[:]
        c = g[:, _H:2 * _H] * cs[:] + g[:, 0:_H] * g[:, 2 * _H:3 * _H]
        cs[:] = c
        tc_s[:] = jnp.tanh(c)

    @pl.when(k == _K - 1)
    def _update():
        h = gates[:, 3 * _H:4 * _H] * tc_s[:]
        hs[:] = h
        hb[:] = h.astype(jnp.bfloat16)

    @pl.when(jnp.logical_and(t == T - 1, k == _K - 1))
    def _head():
        hw = hs[:] * wfc_ref[:]
        r = jax.lax.broadcasted_iota(jnp.int32, (_H, _N), 0) // 128
        m = jax.lax.broadcasted_iota(jnp.int32, (_H, _N), 1)
        seg = (r == m).astype(jnp.float32)
        out_ref[:] = jnp.dot(hw, seg,
                             preferred_element_type=jnp.float32) + bfc_ref[:]


def _lstm_call(P, gb, Whb, wfc_t, bfc):
    B = 16
    T = 50
    return pl.pallas_call(
        _lstm_kernel,
        grid=(T, _K),
        in_specs=[
            pl.BlockSpec((B, _G), lambda t, k: (t, 0)),
            pl.BlockSpec((1, _G), lambda t, k: (0, 0)),
            pl.BlockSpec((1, _H, _RT), lambda t, k: (k, 0, 0)),
            pl.BlockSpec((1, _H), lambda t, k: (0, 0)),
            pl.BlockSpec((1, 1), lambda t, k: (0, 0)),
        ],
        out_specs=pl.BlockSpec((B, _N), lambda t, k: (0, 0)),
        out_shape=jax.ShapeDtypeStruct((B, _N), jnp.float32),
        scratch_shapes=[
            pltpu.VMEM((B, _G), jnp.float32),
            pltpu.VMEM((B, _H), jnp.float32),
            pltpu.VMEM((B, _H), jnp.float32),
            pltpu.VMEM((B, _H), jnp.bfloat16),
            pltpu.VMEM((B, _H), jnp.float32),
        ],
        compiler_params=pltpu.CompilerParams(
            dimension_semantics=("arbitrary", "arbitrary")),
        interpret=_INTERPRET,
    )(P, gb, Whb, wfc_t, bfc)


def _lstm2_kernel(cid_ref, p_ref, gb_ref, whp_ref, whs_ref, wfc_ref, bfc_ref,
                  out_ref, gates, cs, hs, hbf, tc_s, send_sem, recv_sem):
    t = pl.program_id(0)
    k = pl.program_id(1)
    T = pl.num_programs(0)
    cid = cid_ref[0]
    peer = 1 - cid
    HH = _H // 2

    @pl.when(jnp.logical_and(t == 0, k == 0))
    def _init():
        bar = pltpu.get_barrier_semaphore()
        pltpu.semaphore_signal(bar, 1, device_id=peer,
                               device_id_type=pltpu.DeviceIdType.LOGICAL)
        pltpu.semaphore_wait(bar, 1)
        cs[:] = jnp.zeros_like(cs)
        hbf[:] = jnp.zeros_like(hbf)

    def _gate(wh):
        mm = (jax.lax.dot_general(hbf[0], wh[0:HH], (((1,), (0,)), ((), ())),
                                  preferred_element_type=jnp.float32)
              + jax.lax.dot_general(hbf[1], wh[HH:2 * HH],
                                    (((1,), (0,)), ((), ())),
                                    preferred_element_type=jnp.float32))
        sl = pl.ds(k * HH, HH)
        pre = mm + p_ref[:, sl] + gb_ref[:, sl]
        al = jnp.where(k == 2, 1.0, 0.5).astype(jnp.float32)
        ga = jnp.where(k == 2, 0.0, 0.5).astype(jnp.float32)
        gates[:, sl] = al * jnp.tanh(al * pre) + ga

    @pl.when(k == 0)
    def _g0():
        _gate(whp_ref[0])

    @pl.when(k > 0)
    def _gk():
        _gate(whs_ref[0])

    @pl.when(k == 2)
    def _cell():
        g = gates[:]
        c = g[:, HH:2 * HH] * cs[:] + g[:, 0:HH] * g[:, 2 * HH:3 * HH]
        cs[:] = c
        tc_s[:] = jnp.tanh(c)

    @pl.when(k == 3)
    def _update():
        h = gates[:, 3 * HH:4 * HH] * tc_s[:]
        hs[:] = h
        hbf[cid] = h.astype(jnp.bfloat16)

        @pl.when(t < T - 1)
        def _exchange():
            rc = pltpu.make_async_remote_copy(
                hbf.at[cid], hbf.at[cid], send_sem, recv_sem,
                device_id=peer,
                device_id_type=pltpu.DeviceIdType.LOGICAL)
            rc.start()
            rc.wait_send()
            rc.wait_recv()

    @pl.when(jnp.logical_and(t == T - 1, k == 3))
    def _head():
        hw = hs[:] * wfc_ref[:]
        r = jax.lax.broadcasted_iota(jnp.int32, (HH, 16), 0) // 128
        m = jax.lax.broadcasted_iota(jnp.int32, (HH, 16), 1)
        seg = (r == m).astype(jnp.float32)
        out_ref[:] = jnp.dot(hw, seg,
                             preferred_element_type=jnp.float32) + bfc_ref[:]


def _lstm2_local(cid, Pl, gbl, Whp, Whs, wfcl, bfc):
    B = 16
    T = 50
    GH = _G // 2
    return pl.pallas_call(
        _lstm2_kernel,
        grid=(T, 4),
        in_specs=[
            pl.BlockSpec(memory_space=pltpu.SMEM),
            pl.BlockSpec((B, GH), lambda t, k: (t, 0)),
            pl.BlockSpec((1, GH), lambda t, k: (0, 0)),
            pl.BlockSpec((1, _H, _H // 2),
                         lambda t, k: (0, 0, 0)),
            pl.BlockSpec((1, _H, _H // 2),
                         lambda t, k: (jnp.maximum(k - 1, 0), 0, 0)),
            pl.BlockSpec((1, _H // 2), lambda t, k: (0, 0)),
            pl.BlockSpec((1, 1), lambda t, k: (0, 0)),
        ],
        out_specs=pl.BlockSpec((B, 16), lambda t, k: (0, 0)),
        out_shape=jax.ShapeDtypeStruct((B, 16), jnp.float32),
        scratch_shapes=[
            pltpu.VMEM((B, GH), jnp.float32),
            pltpu.VMEM((B, _H // 2), jnp.float32),
            pltpu.VMEM((B, _H // 2), jnp.float32),
            pltpu.VMEM((2, B, _H // 2), jnp.bfloat16),
            pltpu.VMEM((B, _H // 2), jnp.float32),
            pltpu.SemaphoreType.DMA,
            pltpu.SemaphoreType.DMA,
        ],
        compiler_params=pltpu.CompilerParams(
            dimension_semantics=("arbitrary", "arbitrary"),
            collective_id=0),
        interpret=_INTERPRET,
    )(cid, Pl, gbl, Whp, Whs, wfcl, bfc)


def _lstm2_call(Pbig, gb, Wh5, wfc_t, bfc, mesh):
    def local_fn(Pl, gbl, Whpl, Whsl, wfcl, bfcl):
        cid = lax.axis_index("x").astype(jnp.int32).reshape(1)
        return _lstm2_local(cid, Pl, gbl, Whpl[0], Whsl[0], wfcl, bfcl)

    return shard_map(
        local_fn, mesh=mesh,
        in_specs=(P_(None, "x"), P_(None, "x"), P_("x"), P_("x"),
                  P_(None, "x"), P_(None, None)),
        out_specs=P_(None, "x"),
        check_rep=False,
    )(Pbig, gb, Wh5[:, 0:1], Wh5[:, 1:4], wfc_t, bfc)


def _rep(mesh, fn, nout):
    outs = tuple(P_() for _ in range(nout))

    def wrap(*args):
        return shard_map(fn, mesh=mesh,
                         in_specs=tuple(P_() for _ in args),
                         out_specs=outs if nout > 1 else P_(),
                         check_rep=False)(*args)

    return wrap


def kernel(x, edge_index, ln_w, ln_b, W1, b1, W2, b2, Wi, Wh, bi, bh, Wfc, bfc):
    B, T, N, F = x.shape

    devs = jax.devices()
    two_core = len(devs) >= 2 and devs[0].platform == "tpu"
    if two_core:
        mesh = Mesh(np.array(devs[:2]), ("x",))
        graph_c = _rep(mesh, _graph_call, 3)
        ln_c = _rep(mesh, _ln_call, 1)
        mix_c = _rep(mesh, _mix_call, 1)
        proj_c = _rep(mesh, lambda y, wi: _proj_call(y, wi, True), 1)
    else:
        mesh = None
        graph_c, ln_c, mix_c = _graph_call, _ln_call, _mix_call
        proj_c = lambda y, wi: _proj_call(y, wi, False)

    A2, W12, Cmat = graph_c(edge_index, W1, b1.reshape(1, 64),
                            W2, b2.reshape(1, 64))

    xT = jnp.transpose(x, (1, 0, 2, 3)).reshape(T * B * N, F)
    Z = ln_c(xT, ln_w.reshape(1, F), ln_b.reshape(1, F), W12)

    Z2 = Z.reshape(T * B, N, 64).transpose(1, 0, 2).reshape(N, T * B * 64)
    Y2 = mix_c(A2, Z2)
    Yflat = Y2.reshape(N, T * B, 64).transpose(1, 0, 2).reshape(T * B, N * 64)

    cvec = Cmat.reshape(1, N * 64)
    Ybig = jnp.concatenate(
        [Yflat, cvec, jnp.zeros((7, N * 64), jnp.float32)], axis=0)

    bsum = bi.reshape(1, _G) + bh.reshape(1, _G)
    wfc_t = jnp.tile(Wfc, (1, N))
    bfc2 = bfc.reshape(1, 1)

    if two_core:
        Pbig = proj_c(Ybig, Wi)
        gbp = bsum.reshape(4, 2, _H // 2).transpose(1, 0, 2).reshape(1, _G)
        gb = Pbig[T * B:T * B + 1] + gbp
        Wh5 = jnp.transpose(
            Wh.astype(jnp.bfloat16).reshape(4, 2, _H // 2, _H), (1, 0, 3, 2))
        out = _lstm2_call(Pbig, gb, Wh5, wfc_t, bfc2, mesh)
    else:
        Pbig = proj_c(Ybig, Wi)
        gb = Pbig[T * B:T * B + 1] + bsum
        Whb = jnp.transpose(Wh.astype(jnp.bfloat16).reshape(_K, _RT, _H),
                            (0, 2, 1))
        out = _lstm_call(Pbig, gb, Whb, wfc_t, bfc2)
    return out.reshape(B, N, 1)


# speedup vs baseline: 1.7710x; 1.1239x over previous
import functools

import jax
import jax.numpy as jnp
import numpy as np
from jax import lax
from jax.experimental import pallas as pl
from jax.experimental.pallas import tpu as pltpu
from jax.experimental.shard_map import shard_map
from jax.sharding import Mesh, NamedSharding, PartitionSpec as P_

_INTERPRET = False

_N = 32
_E = 160
_H = 4096
_G = 4 * _H
_RT = 2048
_K = _G // _RT


def _graph_kernel(ei_ref, w1_ref, b1_ref, w2_ref, b2_ref,
                  a2_ref, w12_ref, cmat_ref):
    ei = ei_ref[:]
    loop = jax.lax.broadcasted_iota(jnp.int32, (1, _N), 1)
    srcv = jnp.concatenate([ei[0:1, :], loop], axis=1)
    dstv = jnp.concatenate([ei[1:2, :], loop], axis=1)
    nio = jax.lax.broadcasted_iota(jnp.int32, (_N, _E), 0)
    S = (jnp.broadcast_to(srcv, (_N, _E)) == nio).astype(jnp.float32)
    D = (jnp.broadcast_to(dstv, (_N, _E)) == nio).astype(jnp.float32)
    deg = jnp.sum(D, axis=1, keepdims=True)
    dinv = jax.lax.rsqrt(deg)
    wsrc = jnp.sum(S * dinv, axis=0, keepdims=True)
    wdst = jnp.sum(D * dinv, axis=0, keepdims=True)
    Dw = D * (wsrc * wdst)
    A = jax.lax.dot_general(Dw, S, (((1,), (1,)), ((), ())),
                            preferred_element_type=jnp.float32)
    a2_ref[:] = jnp.dot(A, A, preferred_element_type=jnp.float32)
    w12 = jnp.dot(w1_ref[:], w2_ref[:], preferred_element_type=jnp.float32)
    w12_ref[:] = w12
    arow = jnp.sum(A, axis=1, keepdims=True)
    c1 = jnp.dot(b1_ref[:], w2_ref[:], preferred_element_type=jnp.float32)
    cmat_ref[:] = arow * c1 + b2_ref[:]


def _graph_call(edge_index, W1, b1, W2, b2):
    return pl.pallas_call(
        _graph_kernel,
        out_shape=(
            jax.ShapeDtypeStruct((_N, _N), jnp.float32),
            jax.ShapeDtypeStruct((8, 64), jnp.float32),
            jax.ShapeDtypeStruct((_N, 64), jnp.float32),
        ),
        interpret=_INTERPRET,
    )(edge_index, W1, b1, W2, b2)


def _ln_kernel(x_ref, w_ref, b_ref, w12_ref, z_ref):
    xb = x_ref[:]
    mu = jnp.mean(xb, axis=1, keepdims=True)
    var = jnp.mean((xb - mu) ** 2, axis=1, keepdims=True)
    ln = (xb - mu) * jax.lax.rsqrt(var + 1e-5) * w_ref[:] + b_ref[:]
    z_ref[:] = jnp.dot(ln, w12_ref[:], preferred_element_type=jnp.float32)


def _ln_call(x2d, ln_w, ln_b, W12):
    rows = x2d.shape[0]
    R = 1600
    return pl.pallas_call(
        _ln_kernel,
        grid=(rows // R,),
        in_specs=[
            pl.BlockSpec((R, 8), lambda i: (i, 0)),
            pl.BlockSpec((1, 8), lambda i: (0, 0)),
            pl.BlockSpec((1, 8), lambda i: (0, 0)),
            pl.BlockSpec((8, 64), lambda i: (0, 0)),
        ],
        out_specs=pl.BlockSpec((R, 64), lambda i: (i, 0)),
        out_shape=jax.ShapeDtypeStruct((rows, 64), jnp.float32),
        interpret=_INTERPRET,
    )(x2d, ln_w, ln_b, W12)


def _mix_kernel(a2_ref, z_ref, y_ref):
    y_ref[:] = jnp.dot(a2_ref[:], z_ref[:],
                       preferred_element_type=jnp.float32)


def _mix_call(A2, Z2):
    cols = Z2.shape[1]
    C = 6400
    return pl.pallas_call(
        _mix_kernel,
        grid=(cols // C,),
        in_specs=[
            pl.BlockSpec((_N, _N), lambda i: (0, 0)),
            pl.BlockSpec((_N, C), lambda i: (0, i)),
        ],
        out_specs=pl.BlockSpec((_N, C), lambda i: (0, i)),
        out_shape=jax.ShapeDtypeStruct((_N, cols), jnp.float32),
        interpret=_INTERPRET,
    )(A2, Z2)


def _proj_kernel(y_ref, wi_ref, p_ref):
    yb = y_ref[:].astype(jnp.bfloat16)
    wb = wi_ref[:].astype(jnp.bfloat16)
    p_ref[:] = jax.lax.dot_general(yb, wb, (((1,), (1,)), ((), ())),
                                   preferred_element_type=jnp.float32)


def _proj_call(Ybig, Wi, permute):
    rows = Ybig.shape[0]
    RT = 1024
    if permute:
        omap = lambda k: (0, (k % 4 // 2) * 8 + (k // 4) * 2 + k % 2)
    else:
        omap = lambda k: (0, k)
    return pl.pallas_call(
        _proj_kernel,
        grid=(_G // RT,),
        in_specs=[
            pl.BlockSpec((rows, 2048), lambda k: (0, 0)),
            pl.BlockSpec((RT, 2048), lambda k: (k, 0)),
        ],
        out_specs=pl.BlockSpec((rows, RT), omap),
        out_shape=jax.ShapeDtypeStruct((rows, _G), jnp.float32),
        interpret=_INTERPRET,
    )(Ybig, Wi)


def _lstm_kernel(p_ref, gb_ref, wh_ref, wfc_ref, bfc_ref,
                 out_ref, gates, cs, hs, hb, tc_s):
    t = pl.program_id(0)
    k = pl.program_id(1)
    T = pl.num_programs(0)

    @pl.when(jnp.logical_and(t == 0, k == 0))
    def _init():
        cs[:] = jnp.zeros_like(cs)
        hb[:] = jnp.zeros_like(hb)

    mm = jax.lax.dot_general(hb[:], wh_ref[0], (((1,), (0,)), ((), ())),
                             preferred_element_type=jnp.float32)
    sl = pl.ds(k * _RT, _RT)
    pre = mm + p_ref[:, sl] + gb_ref[:, sl]
    is_g = jnp.logical_and(k >= 4, k < 6)
    al = jnp.where(is_g, 1.0, 0.5).astype(jnp.float32)
    ga = jnp.where(is_g, 0.0, 0.5).astype(jnp.float32)
    gates[:, sl] = al * jnp.tanh(al * pre) + ga

    @pl.when(k == 5)
    def _cell():
        g = gates[:]
        c = g[:, _H:2 * _H] * cs[:] + g[:, 0:_H] * g[:, 2 * _H:3 * _H]
        cs[:] = c
        tc_s[:] = jnp.tanh(c)

    @pl.when(k == _K - 1)
    def _update():
        h = gates[:, 3 * _H:4 * _H] * tc_s[:]
        hs[:] = h
        hb[:] = h.astype(jnp.bfloat16)

    @pl.when(jnp.logical_and(t == T - 1, k == _K - 1))
    def _head():
        hw = hs[:] * wfc_ref[:]
        r = jax.lax.broadcasted_iota(jnp.int32, (_H, _N), 0) // 128
        m = jax.lax.broadcasted_iota(jnp.int32, (_H, _N), 1)
        seg = (r == m).astype(jnp.float32)
        out_ref[:] = jnp.dot(hw, seg,
                             preferred_element_type=jnp.float32) + bfc_ref[:]


def _lstm_call(P, gb, Whb, wfc_t, bfc):
    B = 16
    T = 50
    return pl.pallas_call(
        _lstm_kernel,
        grid=(T, _K),
        in_specs=[
            pl.BlockSpec((B, _G), lambda t, k: (t, 0)),
            pl.BlockSpec((1, _G), lambda t, k: (0, 0)),
            pl.BlockSpec((1, _H, _RT), lambda t, k: (k, 0, 0)),
            pl.BlockSpec((1, _H), lambda t, k: (0, 0)),
            pl.BlockSpec((1, 1), lambda t, k: (0, 0)),
        ],
        out_specs=pl.BlockSpec((B, _N), lambda t, k: (0, 0)),
        out_shape=jax.ShapeDtypeStruct((B, _N), jnp.float32),
        scratch_shapes=[
            pltpu.VMEM((B, _G), jnp.float32),
            pltpu.VMEM((B, _H), jnp.float32),
            pltpu.VMEM((B, _H), jnp.float32),
            pltpu.VMEM((B, _H), jnp.bfloat16),
            pltpu.VMEM((B, _H), jnp.float32),
        ],
        compiler_params=pltpu.CompilerParams(
            dimension_semantics=("arbitrary", "arbitrary")),
        interpret=_INTERPRET,
    )(P, gb, Whb, wfc_t, bfc)


def _lstm2_kernel(cid_ref, p_ref, gb_ref, whp_ref, whs_ref, wfc_ref, bfc_ref,
                  out_ref, gates, cs, hs, hbf, tc_s, send_sem, recv_sem):
    t = pl.program_id(0)
    k = pl.program_id(1)
    T = pl.num_programs(0)
    cid = cid_ref[0]
    peer = 1 - cid
    HH = _H // 2

    @pl.when(jnp.logical_and(t == 0, k == 0))
    def _init():
        bar = pltpu.get_barrier_semaphore()
        pltpu.semaphore_signal(bar, 1, device_id=peer,
                               device_id_type=pltpu.DeviceIdType.LOGICAL)
        pltpu.semaphore_wait(bar, 1)
        cs[:] = jnp.zeros_like(cs)
        hbf[:] = jnp.zeros_like(hbf)

    def _gate(wh, col, al, ga):
        mm = (jax.lax.dot_general(hbf[0], wh[0:HH], (((1,), (0,)), ((), ())),
                                  preferred_element_type=jnp.float32)
              + jax.lax.dot_general(hbf[1], wh[HH:2 * HH],
                                    (((1,), (0,)), ((), ())),
                                    preferred_element_type=jnp.float32))
        sl = pl.ds(col, wh.shape[1])
        pre = mm + p_ref[:, sl] + gb_ref[:, sl]
        gates[:, sl] = al * jnp.tanh(al * pre) + ga

    @pl.when(k == 0)
    def _g0():
        _gate(whp_ref[0], 0, 0.5, 0.5)

    @pl.when(k == 1)
    def _g1():
        _gate(whp_ref[1], HH, 0.5, 0.5)

    @pl.when(k >= 2)
    def _gk():
        al = jnp.where(k < 4, 1.0, 0.5).astype(jnp.float32)
        ga = jnp.where(k < 4, 0.0, 0.5).astype(jnp.float32)
        _gate(whs_ref[0], 2 * HH + (k - 2) * (HH // 2), al, ga)

    @pl.when(k == 3)
    def _cell():
        g = gates[:]
        c = g[:, HH:2 * HH] * cs[:] + g[:, 0:HH] * g[:, 2 * HH:3 * HH]
        cs[:] = c
        tc_s[:] = jnp.tanh(c)

    @pl.when(k == 5)
    def _update():
        h = gates[:, 3 * HH:4 * HH] * tc_s[:]
        hs[:] = h
        hbf[cid] = h.astype(jnp.bfloat16)

        @pl.when(t < T - 1)
        def _exchange():
            rc = pltpu.make_async_remote_copy(
                hbf.at[cid], hbf.at[cid], send_sem, recv_sem,
                device_id=peer,
                device_id_type=pltpu.DeviceIdType.LOGICAL)
            rc.start()
            rc.wait_send()
            rc.wait_recv()

    @pl.when(jnp.logical_and(t == T - 1, k == 5))
    def _head():
        hw = hs[:] * wfc_ref[:]
        r = jax.lax.broadcasted_iota(jnp.int32, (HH, 16), 0) // 128
        m = jax.lax.broadcasted_iota(jnp.int32, (HH, 16), 1)
        seg = (r == m).astype(jnp.float32)
        out_ref[:] = jnp.dot(hw, seg,
                             preferred_element_type=jnp.float32) + bfc_ref[:]


def _lstm2_local(cid, Pl, gbl, Whp, Whs, wfcl, bfc):
    B = 16
    T = 50
    GH = _G // 2
    return pl.pallas_call(
        _lstm2_kernel,
        grid=(T, 6),
        in_specs=[
            pl.BlockSpec(memory_space=pltpu.SMEM),
            pl.BlockSpec((B, GH), lambda t, k: (t, 0)),
            pl.BlockSpec((1, GH), lambda t, k: (0, 0)),
            pl.BlockSpec((2, _H, _H // 2),
                         lambda t, k: (0, 0, 0)),
            pl.BlockSpec((1, _H, _H // 4),
                         lambda t, k: (jnp.maximum(k - 2, 0), 0, 0)),
            pl.BlockSpec((1, _H // 2), lambda t, k: (0, 0)),
            pl.BlockSpec((1, 1), lambda t, k: (0, 0)),
        ],
        out_specs=pl.BlockSpec((B, 16), lambda t, k: (0, 0)),
        out_shape=jax.ShapeDtypeStruct((B, 16), jnp.float32),
        scratch_shapes=[
            pltpu.VMEM((B, GH), jnp.float32),
            pltpu.VMEM((B, _H // 2), jnp.float32),
            pltpu.VMEM((B, _H // 2), jnp.float32),
            pltpu.VMEM((2, B, _H // 2), jnp.bfloat16),
            pltpu.VMEM((B, _H // 2), jnp.float32),
            pltpu.SemaphoreType.DMA,
            pltpu.SemaphoreType.DMA,
        ],
        compiler_params=pltpu.CompilerParams(
            dimension_semantics=("arbitrary", "arbitrary"),
            collective_id=0),
        interpret=_INTERPRET,
    )(cid, Pl, gbl, Whp, Whs, wfcl, bfc)


def _lstm2_call(Pbig, gb, Whp, Whs, wfc_t, bfc, mesh):
    def local_fn(Pl, gbl, Whpl, Whsl, wfcl, bfcl):
        cid = lax.axis_index("x").astype(jnp.int32).reshape(1)
        return _lstm2_local(cid, Pl, gbl, Whpl[0], Whsl[0], wfcl, bfcl)

    return shard_map(
        local_fn, mesh=mesh,
        in_specs=(P_(None, "x"), P_(None, "x"), P_("x"), P_("x"),
                  P_(None, "x"), P_(None, None)),
        out_specs=P_(None, "x"),
        check_rep=False,
    )(Pbig, gb, Whp, Whs, wfc_t, bfc)


def _rep(mesh, fn, nout):
    outs = tuple(P_() for _ in range(nout))

    def wrap(*args):
        return shard_map(fn, mesh=mesh,
                         in_specs=tuple(P_() for _ in args),
                         out_specs=outs if nout > 1 else P_(),
                         check_rep=False)(*args)

    return wrap


def kernel(x, edge_index, ln_w, ln_b, W1, b1, W2, b2, Wi, Wh, bi, bh, Wfc, bfc):
    B, T, N, F = x.shape

    devs = jax.devices()
    two_core = len(devs) >= 2 and devs[0].platform == "tpu"
    if two_core:
        mesh = Mesh(np.array(devs[:2]), ("x",))
        graph_c = _rep(mesh, _graph_call, 3)
        ln_c = _rep(mesh, _ln_call, 1)
        mix_c = _rep(mesh, _mix_call, 1)
        proj_c = _rep(mesh, lambda y, wi: _proj_call(y, wi, True), 1)
    else:
        mesh = None
        graph_c, ln_c, mix_c = _graph_call, _ln_call, _mix_call
        proj_c = lambda y, wi: _proj_call(y, wi, False)

    A2, W12, Cmat = graph_c(edge_index, W1, b1.reshape(1, 64),
                            W2, b2.reshape(1, 64))

    xT = jnp.transpose(x, (1, 0, 2, 3)).reshape(T * B * N, F)
    Z = ln_c(xT, ln_w.reshape(1, F), ln_b.reshape(1, F), W12)

    Z2 = Z.reshape(T * B, N, 64).transpose(1, 0, 2).reshape(N, T * B * 64)
    Y2 = mix_c(A2, Z2)
    Yflat = Y2.reshape(N, T * B, 64).transpose(1, 0, 2).reshape(T * B, N * 64)

    cvec = Cmat.reshape(1, N * 64)
    Ybig = jnp.concatenate(
        [Yflat, cvec, jnp.zeros((7, N * 64), jnp.float32)], axis=0)

    bsum = bi.reshape(1, _G) + bh.reshape(1, _G)
    wfc_t = jnp.tile(Wfc, (1, N))
    bfc2 = bfc.reshape(1, 1)

    if two_core:
        Pbig = proj_c(Ybig, Wi)
        gbp = bsum.reshape(4, 2, _H // 2).transpose(1, 0, 2).reshape(1, _G)
        gb = Pbig[T * B:T * B + 1] + gbp
        Whb = Wh.astype(jnp.bfloat16)
        Whp = jnp.transpose(
            Whb.reshape(4, 2, _H // 2, _H)[0:2], (1, 0, 3, 2))
        Whs = jnp.transpose(
            Whb.reshape(4, 2, 2, _H // 4, _H)[2:4],
            (1, 0, 2, 4, 3)).reshape(2, 4, _H, _H // 4)
        out = _lstm2_call(Pbig, gb, Whp, Whs, wfc_t, bfc2, mesh)
    else:
        Pbig = proj_c(Ybig, Wi)
        gb = Pbig[T * B:T * B + 1] + bsum
        Whb = jnp.transpose(Wh.astype(jnp.bfloat16).reshape(_K, _RT, _H),
                            (0, 2, 1))
        out = _lstm_call(Pbig, gb, Whb, wfc_t, bfc2)
    return out.reshape(B, N, 1)
